# baseline (speedup 1.0000x reference)
import jax
import jax.numpy as jnp
from jax.experimental import pallas as pl
from jax.experimental.pallas import tpu as pltpu
from jax.experimental.pallas import tpu_sc as plsc

_SPLIT = 2
_W = 64
_IDX_BLK = 4096
_NBUF = 4
_NTILES = 32


def kernel(indices, weight):
    B, L = indices.shape
    V, D = weight.shape
    Ds = D // _SPLIT
    N = B * L
    NS = N * _SPLIT

    w_split = weight.reshape(V, _SPLIT, Ds).swapaxes(0, 1).reshape(_SPLIT * V, Ds)
    idx2 = (indices.reshape(N, 1)
            + jnp.arange(_SPLIT, dtype=indices.dtype) * V).reshape(NS)

    rows_per_tile = NS // _NTILES
    blocks_per_tile = rows_per_tile // _IDX_BLK
    wpb = _IDX_BLK // _W

    mesh = plsc.VectorSubcoreMesh(core_axis_name="core",
                                  subcore_axis_name="subcore")

    @pl.kernel(
        out_type=jax.ShapeDtypeStruct((NS, Ds), weight.dtype),
        mesh=mesh,
        scratch_types=[
            pltpu.VMEM((2, _IDX_BLK), jnp.int32),
            pltpu.VMEM((_NBUF, _W, Ds), weight.dtype),
            pltpu.SemaphoreType.DMA,
            pltpu.SemaphoreType.DMA,
            pltpu.SemaphoreType.DMA,
            pltpu.SemaphoreType.DMA,
            pltpu.SemaphoreType.DMA,
            pltpu.SemaphoreType.DMA,
            pltpu.SemaphoreType.DMA,
            pltpu.SemaphoreType.DMA,
            pltpu.SemaphoreType.DMA,
            pltpu.SemaphoreType.DMA,
        ],
    )
    def sc_gather(i_hbm, w_hbm, o_hbm, idxb, rows,
                  isem0, isem1, g0, g1, g2, g3, w0, w1, w2, w3):
        isems = [isem0, isem1]
        gsems = [g0, g1, g2, g3]
        wsems = [w0, w1, w2, w3]

        wid = (jax.lax.axis_index("subcore") * 2
               + jax.lax.axis_index("core"))
        base = wid * rows_per_tile

        def wait_write(b):
            pltpu.make_async_copy(rows.at[b], o_hbm.at[pl.ds(base, _W)],
                                  wsems[b]).wait()

        def start_gather(p, k, b):
            pltpu.async_copy(w_hbm.at[idxb.at[p, pl.ds(k * _W, _W)]],
                             rows.at[b], gsems[b])

        def wait_gather(p, k, b):
            pltpu.make_async_copy(
                w_hbm.at[idxb.at[p, pl.ds(k * _W, _W)]],
                rows.at[b], gsems[b]).wait()

        pltpu.async_copy(i_hbm.at[pl.ds(base, _IDX_BLK)], idxb.at[0],
                         isems[0])

        @pl.loop(0, blocks_per_tile, step=2)
        def _(g):
            for p in range(2):
                blk = g + p
                blk_base = base + blk * _IDX_BLK

                pltpu.make_async_copy(
                    i_hbm.at[pl.ds(blk_base, _IDX_BLK)], idxb.at[p],
                    isems[p]).wait()

                @pl.when(blk + 1 < blocks_per_tile)
                def _():
                    pltpu.async_copy(
                        i_hbm.at[pl.ds(blk_base + _IDX_BLK, _IDX_BLK)],
                        idxb.at[1 - p], isems[1 - p])

                for b in range(_NBUF - 1):
                    @pl.when(blk > 0)
                    def _():
                        wait_write(b)
                    start_gather(p, b, b)

                @pl.loop(0, wpb, step=_NBUF)
                def _(kk):
                    for b in range(_NBUF):
                        k = kk + b
                        row0 = blk_base + k * _W

                        wait_gather(p, k, b)
                        pltpu.async_copy(rows.at[b],
                                         o_hbm.at[pl.ds(row0, _W)],
                                         wsems[b])

                        @pl.when(k + _NBUF - 1 < wpb)
                        def _():
                            bn = (b + _NBUF - 1) % _NBUF
                            @pl.when(blk * wpb + k > 0)
                            def _():
                                wait_write(bn)
                            start_gather(p, k + _NBUF - 1, bn)

        for b in range(_NBUF):
            wait_write(b)

    out = sc_gather(idx2, w_split)
    return out.reshape(B, L, D)


# speedup vs baseline: 4.5790x; 4.5790x over previous
import jax
import jax.numpy as jnp
from jax.experimental import pallas as pl
from jax.experimental.pallas import tpu as pltpu
from jax.experimental.pallas import tpu_sc as plsc

_W = 64
_PAD = 128
_NTILES = 32


def kernel(indices, weight):
    B, L = indices.shape
    V, D = weight.shape
    N = B * L

    idx_pad = jnp.pad(indices.reshape(N // _W, _W),
                      ((0, 0), (0, _PAD - _W)))

    n_win = N // _W
    wpt = n_win // _NTILES

    mesh = plsc.VectorSubcoreMesh(core_axis_name="core",
                                  subcore_axis_name="subcore")

    @pl.kernel(out_type=jax.ShapeDtypeStruct((N, D), weight.dtype), mesh=mesh)
    def sc_gather(w_hbm, i_hbm, o_hbm):
        def body(i_vmem, o_vmem):
            pltpu.sync_copy(w_hbm.at[i_vmem.at[0, pl.ds(0, _W)]], o_vmem)

        pltpu.emit_pipeline(
            body,
            grid=(_NTILES, wpt),
            in_specs=[pl.BlockSpec((1, _PAD),
                                   index_map=lambda c, i: (c * wpt + i, 0))],
            out_specs=[pl.BlockSpec((_W, D),
                                    index_map=lambda c, i: (c * wpt + i, 0))],
            core_axis_name=("core", "subcore"),
            dimension_semantics=(pltpu.PARALLEL, pltpu.ARBITRARY),
        )(i_hbm, o_hbm)

    out = sc_gather(weight, idx_pad)
    return out.reshape(B, L, D)
